# Initial kernel scaffold; baseline (speedup 1.0000x reference)
import functools

import jax
import jax.numpy as jnp
from jax.experimental import pallas as pl
from jax.experimental.pallas import tpu as pltpu

N = 100000
NUM_GRAPHS = 512


def _mm_body(x_ref, w_ref, b_ref, o_ref):
    o_ref[...] = (
        jnp.dot(x_ref[...], w_ref[...], preferred_element_type=jnp.float32)
        + b_ref[...]
    )


def _matmul_bias(x, W, b, block_rows=2000):
    n, fi = x.shape
    fo = W.shape[1]
    grid = n // block_rows
    return pl.pallas_call(
        _mm_body,
        grid=(grid,),
        in_specs=[
            pl.BlockSpec((block_rows, fi), lambda i: (i, 0)),
            pl.BlockSpec((fi, fo), lambda i: (0, 0)),
            pl.BlockSpec((1, fo), lambda i: (0, 0)),
        ],
        out_specs=pl.BlockSpec((block_rows, fo), lambda i: (i, 0)),
        out_shape=jax.ShapeDtypeStruct((n, fo), jnp.float32),
    )(x, W, b.reshape(1, fo))


def _head_body(p_ref, w1, b1, g4, be4, w2, b2, g5, be5, wo, bo, o_ref):
    def bn(h, g, be):
        mu = jnp.mean(h, axis=0, keepdims=True)
        var = jnp.mean((h - mu) ** 2, axis=0, keepdims=True)
        return g * (h - mu) * jax.lax.rsqrt(var + 1e-5) + be

    z = p_ref[...] @ w1[...] + b1[...]
    z = jnp.maximum(bn(z, g4[...], be4[...]), 0.0)
    z = z @ w2[...] + b2[...]
    z = jnp.maximum(bn(z, g5[...], be5[...]), 0.0)
    o_ref[...] = z @ wo[...] + bo[...]


def _head(pooled, p):
    args = (
        pooled,
        p["fc1W"], p["fc1b"].reshape(1, -1), p["g4"].reshape(1, -1),
        p["be4"].reshape(1, -1),
        p["fc2W"], p["fc2b"].reshape(1, -1), p["g5"].reshape(1, -1),
        p["be5"].reshape(1, -1),
        p["foW"], p["fob"].reshape(1, -1),
    )
    return pl.pallas_call(
        _head_body,
        out_shape=jax.ShapeDtypeStruct((NUM_GRAPHS, p["foW"].shape[1]), jnp.float32),
    )(*args)


def _bn(h, gamma, beta, eps=1e-5):
    mu = jnp.mean(h, axis=0)
    var = jnp.var(h, axis=0)
    return gamma * (h - mu) * jax.lax.rsqrt(var + eps) + beta


def kernel(x, edge_index, batch, params):
    p = params
    src, dst = edge_index[0], edge_index[1]
    n = x.shape[0]

    deg = jnp.zeros((n,), jnp.float32).at[dst].add(1.0) + 1.0
    dinv = jax.lax.rsqrt(deg)
    norm = dinv[src] * dinv[dst]

    def prop(g):
        msg = g[src] * norm[:, None]
        out = jnp.zeros_like(g).at[dst].add(msg)
        return out + g * (dinv * dinv)[:, None]

    h = _matmul_bias(prop(x), p["W1"], p["b1"])
    h = jnp.maximum(_bn(h, p["g1"], p["be1"]), 0.0)
    h = prop(_matmul_bias(h, p["W2"], jnp.zeros_like(p["b2"]))) + p["b2"]
    h = jnp.maximum(_bn(h, p["g2"], p["be2"]), 0.0)
    h = prop(_matmul_bias(h, p["W3"], jnp.zeros_like(p["b3"]))) + p["b3"]
    h = jnp.maximum(_bn(h, p["g3"], p["be3"]), 0.0)

    summed = jax.ops.segment_sum(h, batch, num_segments=NUM_GRAPHS)
    cnt = jax.ops.segment_sum(jnp.ones((n,), jnp.float32), batch, num_segments=NUM_GRAPHS)
    pooled = summed / jnp.clip(cnt, 1.0)[:, None]
    return _head(pooled, p)


# speedup vs baseline: 1.0579x; 1.0579x over previous
import functools

import jax
import jax.numpy as jnp
from jax.experimental import pallas as pl
from jax.experimental.pallas import tpu as pltpu

N = 100000
NUM_GRAPHS = 512


def _mm_body(x_ref, w_ref, b_ref, o_ref, *, precision):
    o_ref[...] = (
        jnp.dot(x_ref[...], w_ref[...], preferred_element_type=jnp.float32,
                precision=precision)
        + b_ref[...]
    )


def _matmul_bias(x, W, b, block_rows=2000, precision=None):
    n, fi = x.shape
    fo = W.shape[1]
    grid = n // block_rows
    return pl.pallas_call(
        functools.partial(_mm_body, precision=precision),
        grid=(grid,),
        in_specs=[
            pl.BlockSpec((block_rows, fi), lambda i: (i, 0)),
            pl.BlockSpec((fi, fo), lambda i: (0, 0)),
            pl.BlockSpec((1, fo), lambda i: (0, 0)),
        ],
        out_specs=pl.BlockSpec((block_rows, fo), lambda i: (i, 0)),
        out_shape=jax.ShapeDtypeStruct((n, fo), jnp.float32),
    )(x, W, b.reshape(1, fo))


def _head_body(p_ref, w1, b1, g4, be4, w2, b2, g5, be5, wo, bo, o_ref):
    def bn(h, g, be):
        mu = jnp.mean(h, axis=0, keepdims=True)
        var = jnp.mean((h - mu) ** 2, axis=0, keepdims=True)
        a = var + 1e-5
        r = jax.lax.rsqrt(a)
        r = r * (1.5 - 0.5 * a * r * r)
        return g * (h - mu) * r + be

    mm = jnp.dot
    z = mm(p_ref[...], w1[...]) + b1[...]
    z = jnp.maximum(bn(z, g4[...], be4[...]), 0.0)
    z = mm(z, w2[...]) + b2[...]
    z = jnp.maximum(bn(z, g5[...], be5[...]), 0.0)
    o_ref[...] = mm(z, wo[...]) + bo[...]


def _head(pooled, p):
    args = (
        pooled,
        p["fc1W"], p["fc1b"].reshape(1, -1), p["g4"].reshape(1, -1),
        p["be4"].reshape(1, -1),
        p["fc2W"], p["fc2b"].reshape(1, -1), p["g5"].reshape(1, -1),
        p["be5"].reshape(1, -1),
        p["foW"], p["fob"].reshape(1, -1),
    )
    return pl.pallas_call(
        _head_body,
        out_shape=jax.ShapeDtypeStruct((NUM_GRAPHS, p["foW"].shape[1]), jnp.float32),
    )(*args)


def _bn(h, gamma, beta, eps=1e-5):
    mu = jnp.mean(h, axis=0)
    var = jnp.var(h, axis=0)
    return gamma * (h - mu) * jax.lax.rsqrt(var + eps) + beta


def kernel(x, edge_index, batch, params):
    p = params
    src, dst = edge_index[0], edge_index[1]
    n = x.shape[0]

    deg = jnp.zeros((n,), jnp.float32).at[dst].add(1.0) + 1.0
    dinv = jax.lax.rsqrt(deg)
    norm = dinv[src] * dinv[dst]

    def prop(g):
        msg = g[src] * norm[:, None]
        out = jnp.zeros_like(g).at[dst].add(msg)
        return out + g * (dinv * dinv)[:, None]

    h = prop(_matmul_bias(x, p["W1"], jnp.zeros_like(p["b1"]))) + p["b1"]
    h = jnp.maximum(_bn(h, p["g1"], p["be1"]), 0.0)
    h = prop(_matmul_bias(h, p["W2"], jnp.zeros_like(p["b2"]))) + p["b2"]
    h = jnp.maximum(_bn(h, p["g2"], p["be2"]), 0.0)
    h = prop(_matmul_bias(h, p["W3"], jnp.zeros_like(p["b3"]))) + p["b3"]
    h = jnp.maximum(_bn(h, p["g3"], p["be3"]), 0.0)

    summed = jax.ops.segment_sum(h, batch, num_segments=NUM_GRAPHS)
    cnt = jax.ops.segment_sum(jnp.ones((n,), jnp.float32), batch, num_segments=NUM_GRAPHS)
    pooled = summed / jnp.clip(cnt, 1.0)[:, None]
    return _head(pooled, p)
